# Initial kernel scaffold; baseline (speedup 1.0000x reference)
import jax
import jax.numpy as jnp
from jax.experimental import pallas as pl


def kernel(pre_embedding, table):
    raise NotImplementedError("write your pallas kernel here")


# speedup vs baseline: 1.6958x; 1.6958x over previous
import functools

import jax
import jax.numpy as jnp
from jax import lax
from jax.experimental import pallas as pl
from jax.experimental.pallas import tpu as pltpu
from jax.experimental.pallas import tpu_sc as plsc

_CH = 128


@functools.cache
def _make_sc_gather(B: int, D: int):
    info = plsc.get_sparse_core_info()
    NC, NS = info.num_cores, info.num_subcores
    NW = NC * NS
    assert B % (NW * _CH) == 0
    BPW = B // NW
    NCH = BPW // _CH
    mesh = plsc.VectorSubcoreMesh(core_axis_name="c", subcore_axis_name="s")

    @functools.partial(
        pl.kernel,
        mesh=mesh,
        out_type=jax.ShapeDtypeStruct((B, D), jnp.float32),
        compiler_params=pltpu.CompilerParams(use_tc_tiling_on_sc=False),
        scratch_types=[
            pltpu.VMEM((NCH, _CH), jnp.int32),
            pltpu.VMEM((_CH, D), jnp.float32),
            pltpu.SemaphoreType.DMA,
        ],
    )
    def gather_kernel(idx_hbm, table_hbm, out_hbm, idx_v, rows_v, sem):
        wid = lax.axis_index("s") * NC + lax.axis_index("c")
        pltpu.sync_copy(idx_hbm.at[pl.ds(wid * NCH, NCH)], idx_v)

        def body(j, _):
            pltpu.async_copy(table_hbm.at[idx_v.at[j]], rows_v, sem).wait()
            pltpu.sync_copy(rows_v, out_hbm.at[pl.ds(wid * BPW + j * _CH, _CH)])
            return 0

        lax.fori_loop(0, NCH, body, 0)

    return gather_kernel


def kernel(pre_embedding, table):
    B0, B1 = pre_embedding.shape
    V, D = table.shape
    B = B0 * B1
    idx2d = pre_embedding.reshape(B // _CH, _CH).astype(jnp.int32)
    out = _make_sc_gather(B, D)(idx2d, table)
    return out.reshape(B0, B1, D)


# speedup vs baseline: 1.8786x; 1.1078x over previous
import functools

import jax
import jax.numpy as jnp
from jax import lax
from jax.experimental import pallas as pl
from jax.experimental.pallas import tpu as pltpu
from jax.experimental.pallas import tpu_sc as plsc

_CH = 128
_NBUF = 8
_A = 4


@functools.cache
def _make_sc_gather(B: int, D: int):
    info = plsc.get_sparse_core_info()
    NC, NS = info.num_cores, info.num_subcores
    NW = NC * NS
    assert B % (NW * _CH) == 0
    BPW = B // NW
    NCH = BPW // _CH
    assert NCH > _NBUF and (NCH - _NBUF) % _NBUF == 0
    mesh = plsc.VectorSubcoreMesh(core_axis_name="c", subcore_axis_name="s")

    @functools.partial(
        pl.kernel,
        mesh=mesh,
        out_type=jax.ShapeDtypeStruct((B, D), jnp.float32),
        compiler_params=pltpu.CompilerParams(use_tc_tiling_on_sc=False),
        scratch_types=[
            pltpu.VMEM((NCH, _CH), jnp.int32),
            pltpu.VMEM((_NBUF, _CH, D), jnp.float32),
            pltpu.SemaphoreType.DMA((_NBUF,)),
            pltpu.SemaphoreType.DMA((_NBUF,)),
        ],
    )
    def gather_kernel(idx_hbm, table_hbm, out_hbm, idx_v, rows_v, gsem, osem):
        wid = lax.axis_index("s") * NC + lax.axis_index("c")
        pltpu.sync_copy(idx_hbm.at[pl.ds(wid * NCH, NCH)], idx_v)
        obase = wid * BPW

        def fire_gather(g, b):
            pltpu.async_copy(table_hbm.at[idx_v.at[g]], rows_v.at[b], gsem.at[b])

        def wait_gather(g, b):
            pltpu.make_async_copy(
                table_hbm.at[idx_v.at[g]], rows_v.at[b], gsem.at[b]).wait()

        def fire_out(g, b):
            pltpu.async_copy(
                rows_v.at[b], out_hbm.at[pl.ds(obase + g * _CH, _CH)], osem.at[b])

        def wait_out(g, b):
            pltpu.make_async_copy(
                rows_v.at[b], out_hbm.at[pl.ds(obase + g * _CH, _CH)],
                osem.at[b]).wait()

        for g in range(_A):
            fire_gather(g, g)
        start = _NBUF - _A
        for g in range(start):
            wait_gather(g, g % _NBUF)
            fire_out(g, g % _NBUF)
            fire_gather(g + _A, (g + _A) % _NBUF)

        def outer(i, carry):
            go = start + i * _NBUF
            for b in range(_NBUF):
                s = (start + b) % _NBUF
                g = go + b
                wait_gather(g, s)
                fire_out(g, s)
                s2 = (start + b + _A) % _NBUF
                wait_out(g + _A - _NBUF, s2)
                fire_gather(g + _A, s2)
            return carry

        lax.fori_loop(0, (NCH - _NBUF) // _NBUF, outer, 0)

        for k in range(_A):
            g = NCH - _A + k
            s = (start + k) % _NBUF
            wait_gather(g, s)
            fire_out(g, s)
        for k in range(_NBUF):
            g = NCH - _NBUF + k
            wait_out(g, g % _NBUF)

    return gather_kernel


def kernel(pre_embedding, table):
    B0, B1 = pre_embedding.shape
    V, D = table.shape
    B = B0 * B1
    idx2d = pre_embedding.reshape(B // _CH, _CH).astype(jnp.int32)
    out = _make_sc_gather(B, D)(idx2d, table)
    return out.reshape(B0, B1, D)
